# baseline (speedup 1.0000x reference)
import numpy as np
import jax
import jax.numpy as jnp
from jax import lax
from jax.experimental import pallas as pl
from jax.experimental.pallas import tpu as pltpu
from jax.experimental.pallas import tpu_sc as plsc

_SR = 44100
_SIL_LO = int(0.01 * _SR)
_SIL_HI = int(0.1 * _SR)
_RATIO_LO, _RATIO_HI = 0.1, 0.2
_LENGTH = 441000
_BATCH = 8


def _plan(length):
    rng = np.random.default_rng(0)
    cur = 0
    total_target = int(rng.integers(int(_RATIO_LO * length), int(_RATIO_HI * length)))
    segs = []
    while cur < total_target:
        sl = int(rng.integers(_SIL_LO, _SIL_HI))
        start = int(rng.integers(0, length - sl))
        mode = int(rng.integers(0, 2))
        segs.append((start, sl, mode))
        cur += sl
    return segs


_SEGS = _plan(_LENGTH)

_NW = 32
_SPAN_A = 108 * 128
_SPAN_B = 107 * 128
_NA = 21
_BASE_B0 = _NA * _SPAN_A
_BASE_LAST = _BASE_B0 + 10 * _SPAN_B
_SPAN_LAST = _LENGTH - _BASE_LAST


def _sc_envelope_body(env_hbm, mbuf, sem):
    cid = lax.axis_index("c")
    sid = lax.axis_index("s")
    wid = sid * 2 + cid

    is_a = wid < _NA
    is_last = wid == _NW - 1
    base = jnp.where(
        is_a, wid * _SPAN_A,
        jnp.where(is_last, _BASE_LAST, _BASE_B0 + (wid - _NA) * _SPAN_B))
    span = jnp.where(is_a, _SPAN_A, jnp.where(is_last, _SPAN_LAST, _SPAN_B))

    ones16 = jnp.full((16,), 1.0, jnp.float32)

    def _init(i, carry):
        mbuf[pl.ds(i * 16, 16)] = ones16
        return carry
    lax.fori_loop(0, _SPAN_A // 16, _init, 0, unroll=4)
    @pl.when(is_last)
    def _init_tail():
        mbuf[pl.ds(_SPAN_LAST - 16, 16)] = ones16

    iota = lax.iota(jnp.int32, 16)
    for start, sl, mode in _SEGS:
        half = sl // 2
        m = sl - half
        a_up = np.float32(1.0 / (half - 1))
        a_dn = np.float32(-1.0 / (m - 1))
        ls = jnp.maximum(start - base, 0)
        le = jnp.minimum(start + sl - base, span)

        nch = sl // 16 + 2
        cbase = (ls // 16) * 16

        def _apply_chunk(ci, carry, _start=start, _mode=mode, _half=half,
                         _a_up=a_up, _a_dn=a_dn, _ls=ls, _le=le,
                         _cbase=cbase):
            off = _cbase + ci * 16
            idx = off + iota
            lanes = jnp.logical_and(idx >= _ls, idx < _le)
            pos = jnp.minimum(off, _SPAN_A - 16)
            win = pl.ds(pos, 16)
            cur = mbuf[win]
            if _mode == 0:
                new = jnp.where(lanes, jnp.float32(0.0), cur)
            else:
                k = (idx + base) - _start
                kf = k.astype(jnp.float32)
                fade = jnp.where(k < _half, kf * _a_up,
                                 1.0 + (kf - _half) * _a_dn)
                new = jnp.where(lanes, cur * fade, cur)
            mbuf[win] = new
            return carry

        @pl.when(ls < le)
        def _apply_seg(_nch=nch, _body=_apply_chunk):
            lax.fori_loop(0, _nch, _body, 0)

    @pl.when(is_a)
    def _out_a():
        b = pl.multiple_of(wid * _SPAN_A, 128)
        pltpu.async_copy(mbuf.at[pl.ds(0, _SPAN_A)],
                         env_hbm.at[0, 0, pl.ds(b, _SPAN_A)], sem).wait()

    @pl.when(jnp.logical_and(wid >= _NA, wid < _NW - 1))
    def _out_b():
        b = pl.multiple_of(_BASE_B0 + (wid - _NA) * _SPAN_B, 128)
        pltpu.async_copy(mbuf.at[pl.ds(0, _SPAN_B)],
                         env_hbm.at[0, 0, pl.ds(b, _SPAN_B)], sem).wait()

    @pl.when(is_last)
    def _out_last():
        pltpu.async_copy(mbuf.at[pl.ds(0, _SPAN_LAST)],
                         env_hbm.at[0, 0, pl.ds(_BASE_LAST, _SPAN_LAST)],
                         sem).wait()


def _build_envelope():
    run = pl.kernel(
        _sc_envelope_body,
        out_type=jax.ShapeDtypeStruct((1, 1, _LENGTH), jnp.float32),
        mesh=plsc.VectorSubcoreMesh(core_axis_name="c", subcore_axis_name="s"),
        scratch_types=[
            pltpu.VMEM((_SPAN_A,), jnp.float32),
            pltpu.SemaphoreType.DMA,
        ],
    )
    return run()


_CHUNK = 110592


def _mul_kernel(w_ref, m_ref, o_ref):
    o_ref[...] = w_ref[...] * m_ref[...]


def kernel(waveform):
    b, c, length = waveform.shape
    env = _build_envelope()
    grid = (pl.cdiv(length, _CHUNK),)
    out = pl.pallas_call(
        _mul_kernel,
        grid=grid,
        in_specs=[
            pl.BlockSpec((b, c, _CHUNK), lambda i: (0, 0, i)),
            pl.BlockSpec((1, c, _CHUNK), lambda i: (0, 0, i)),
        ],
        out_specs=pl.BlockSpec((b, c, _CHUNK), lambda i: (0, 0, i)),
        out_shape=jax.ShapeDtypeStruct((b, c, length), jnp.float32),
        compiler_params=pltpu.CompilerParams(
            dimension_semantics=("parallel",),
        ),
    )(waveform, env)
    return out


# speedup vs baseline: 1.0084x; 1.0084x over previous
import numpy as np
import jax
import jax.numpy as jnp
from jax import lax
from jax.experimental import pallas as pl
from jax.experimental.pallas import tpu as pltpu
from jax.experimental.pallas import tpu_sc as plsc

_SR = 44100
_SIL_LO = int(0.01 * _SR)
_SIL_HI = int(0.1 * _SR)
_RATIO_LO, _RATIO_HI = 0.1, 0.2
_LENGTH = 441000
_BATCH = 8


def _plan(length):
    rng = np.random.default_rng(0)
    cur = 0
    total_target = int(rng.integers(int(_RATIO_LO * length), int(_RATIO_HI * length)))
    segs = []
    while cur < total_target:
        sl = int(rng.integers(_SIL_LO, _SIL_HI))
        start = int(rng.integers(0, length - sl))
        mode = int(rng.integers(0, 2))
        segs.append((start, sl, mode))
        cur += sl
    return segs


_SEGS = _plan(_LENGTH)

_NW = 32
_SPAN_A = 108 * 128
_SPAN_B = 107 * 128
_NA = 21
_BASE_B0 = _NA * _SPAN_A
_BASE_LAST = _BASE_B0 + 10 * _SPAN_B
_SPAN_LAST = _LENGTH - _BASE_LAST


def _sc_envelope_body(env_hbm, mbuf, sem):
    cid = lax.axis_index("c")
    sid = lax.axis_index("s")
    wid = sid * 2 + cid

    is_a = wid < _NA
    is_last = wid == _NW - 1
    base = jnp.where(
        is_a, wid * _SPAN_A,
        jnp.where(is_last, _BASE_LAST, _BASE_B0 + (wid - _NA) * _SPAN_B))
    span = jnp.where(is_a, _SPAN_A, jnp.where(is_last, _SPAN_LAST, _SPAN_B))

    ones16 = jnp.full((16,), 1.0, jnp.float32)

    def _init(i, carry):
        mbuf[pl.ds(i * 16, 16)] = ones16
        return carry
    lax.fori_loop(0, _SPAN_A // 16, _init, 0, unroll=4)
    @pl.when(is_last)
    def _init_tail():
        mbuf[pl.ds(_SPAN_LAST - 16, 16)] = ones16

    iota = lax.iota(jnp.int32, 16)
    for start, sl, mode in _SEGS:
        half = sl // 2
        m = sl - half
        a_up = np.float32(1.0 / (half - 1))
        a_dn = np.float32(-1.0 / (m - 1))
        ls = jnp.maximum(start - base, 0)
        le = jnp.minimum(start + sl - base, span)

        nch = sl // 16 + 2
        cbase = (ls // 16) * 16

        def _apply_chunk(ci, carry, _start=start, _mode=mode, _half=half,
                         _a_up=a_up, _a_dn=a_dn, _ls=ls, _le=le,
                         _cbase=cbase):
            off = _cbase + ci * 16
            idx = off + iota
            lanes = jnp.logical_and(idx >= _ls, idx < _le)
            pos = jnp.minimum(off, _SPAN_A - 16)
            win = pl.ds(pos, 16)
            cur = mbuf[win]
            if _mode == 0:
                new = jnp.where(lanes, jnp.float32(0.0), cur)
            else:
                k = (idx + base) - _start
                kf = k.astype(jnp.float32)
                fade = jnp.where(k < _half, kf * _a_up,
                                 1.0 + (kf - _half) * _a_dn)
                new = jnp.where(lanes, cur * fade, cur)
            mbuf[win] = new
            return carry

        @pl.when(ls < le)
        def _apply_seg(_nch=nch, _body=_apply_chunk):
            lax.fori_loop(0, _nch, _body, 0)

    @pl.when(is_a)
    def _out_a():
        b = pl.multiple_of(wid * _SPAN_A, 128)
        pltpu.async_copy(mbuf.at[pl.ds(0, _SPAN_A)],
                         env_hbm.at[0, 0, pl.ds(b, _SPAN_A)], sem).wait()

    @pl.when(jnp.logical_and(wid >= _NA, wid < _NW - 1))
    def _out_b():
        b = pl.multiple_of(_BASE_B0 + (wid - _NA) * _SPAN_B, 128)
        pltpu.async_copy(mbuf.at[pl.ds(0, _SPAN_B)],
                         env_hbm.at[0, 0, pl.ds(b, _SPAN_B)], sem).wait()

    @pl.when(is_last)
    def _out_last():
        pltpu.async_copy(mbuf.at[pl.ds(0, _SPAN_LAST)],
                         env_hbm.at[0, 0, pl.ds(_BASE_LAST, _SPAN_LAST)],
                         sem).wait()


def _build_envelope():
    run = pl.kernel(
        _sc_envelope_body,
        out_type=jax.ShapeDtypeStruct((1, 1, _LENGTH), jnp.float32),
        mesh=plsc.VectorSubcoreMesh(core_axis_name="c", subcore_axis_name="s"),
        scratch_types=[
            pltpu.VMEM((_SPAN_A,), jnp.float32),
            pltpu.SemaphoreType.DMA,
        ],
    )
    return run()


_CHUNK = 147456


def _mul_kernel(w_ref, m_ref, o_ref):
    o_ref[...] = w_ref[...] * m_ref[...]


def kernel(waveform):
    b, c, length = waveform.shape
    env = _build_envelope()
    grid = (pl.cdiv(length, _CHUNK),)
    out = pl.pallas_call(
        _mul_kernel,
        grid=grid,
        in_specs=[
            pl.BlockSpec((b, c, _CHUNK), lambda i: (0, 0, i)),
            pl.BlockSpec((1, c, _CHUNK), lambda i: (0, 0, i)),
        ],
        out_specs=pl.BlockSpec((b, c, _CHUNK), lambda i: (0, 0, i)),
        out_shape=jax.ShapeDtypeStruct((b, c, length), jnp.float32),
        compiler_params=pltpu.CompilerParams(
            dimension_semantics=("parallel",),
        ),
    )(waveform, env)
    return out
